# baseline (speedup 1.0000x reference)
import functools

import jax
import jax.numpy as jnp
from jax import lax
from jax.experimental import pallas as pl
from jax.experimental.pallas import tpu as pltpu

N_NODES = 10000
N_EDGES = 320000
D = 128
H = 3
EMB_ROWS = 22754
NEG_SLOPE = 0.2

EMB_PAD = 23040
N_TILE = 320
N_PAD = 32 * N_TILE


def _prep_body(x_ref, wlin_ref, wh_ref, asrc_ref, adst_ref,
               xlin_ref, sa_ref, bmax_ref):
    i = pl.program_id(0)
    xlin = jnp.dot(x_ref[...], wlin_ref[...], preferred_element_type=jnp.float32)
    xlin_ref[...] = xlin
    cols = []
    for h in range(H):
        cols.append(jnp.dot(wh_ref[h], asrc_ref[h], preferred_element_type=jnp.float32))
    cols.append(jnp.zeros((D,), jnp.float32))
    for h in range(H):
        cols.append(jnp.dot(wh_ref[h], adst_ref[h], preferred_element_type=jnp.float32))
    cols.append(jnp.zeros((D,), jnp.float32))
    u8 = jnp.stack(cols, axis=1)
    sa = jnp.dot(xlin, u8, preferred_element_type=jnp.float32)
    sa_ref[...] = sa
    blk_max = (jnp.max(sa[:, :3]) + jnp.max(sa[:, 4:7])).reshape(1, 1)

    @pl.when(i == 0)
    def _():
        bmax_ref[...] = blk_max

    @pl.when(i > 0)
    def _():
        bmax_ref[...] = jnp.maximum(bmax_ref[...], blk_max)


def _tc_prep(x, W_lin, W_heads, att_src, att_dst):
    blk = 1000
    return pl.pallas_call(
        _prep_body,
        grid=(N_NODES // blk,),
        in_specs=[
            pl.BlockSpec((blk, D), lambda i: (i, 0)),
            pl.BlockSpec((D, D), lambda i: (0, 0)),
            pl.BlockSpec((H, D, D), lambda i: (0, 0, 0)),
            pl.BlockSpec((H, D), lambda i: (0, 0)),
            pl.BlockSpec((H, D), lambda i: (0, 0)),
        ],
        out_specs=[
            pl.BlockSpec((blk, D), lambda i: (i, 0)),
            pl.BlockSpec((blk, 8), lambda i: (i, 0)),
            pl.BlockSpec((1, 1), lambda i: (0, 0)),
        ],
        out_shape=[
            jax.ShapeDtypeStruct((N_NODES, D), jnp.float32),
            jax.ShapeDtypeStruct((N_NODES, 8), jnp.float32),
            jax.ShapeDtypeStruct((1, 1), jnp.float32),
        ],
    )(x, W_lin, W_heads, att_src, att_dst)


def _tscore_body(emb_ref, we_ref, ae_ref, t_ref, tmax_ref):
    i = pl.program_id(0)
    cols = []
    for h in range(H):
        cols.append(jnp.dot(we_ref[h], ae_ref[h], preferred_element_type=jnp.float32))
    cols.append(jnp.zeros((D,), jnp.float32))
    v4 = jnp.stack(cols, axis=1)
    t = jnp.dot(emb_ref[...], v4, preferred_element_type=jnp.float32)
    t_ref[...] = t
    blk_max = jnp.max(t[:, :3]).reshape(1, 1)

    @pl.when(i == 0)
    def _():
        tmax_ref[...] = blk_max

    @pl.when(i > 0)
    def _():
        tmax_ref[...] = jnp.maximum(tmax_ref[...], blk_max)


def _tc_tscore(emb_pad, W_edge, att_edge):
    blk = 512
    return pl.pallas_call(
        _tscore_body,
        grid=(EMB_PAD // blk,),
        in_specs=[
            pl.BlockSpec((blk, D), lambda i: (i, 0)),
            pl.BlockSpec((H, D, D), lambda i: (0, 0, 0)),
            pl.BlockSpec((H, D), lambda i: (0, 0)),
        ],
        out_specs=[
            pl.BlockSpec((blk, 4), lambda i: (i, 0)),
            pl.BlockSpec((1, 1), lambda i: (0, 0)),
        ],
        out_shape=[
            jax.ShapeDtypeStruct((EMB_PAD, 4), jnp.float32),
            jax.ShapeDtypeStruct((1, 1), jnp.float32),
        ],
    )(emb_pad, W_edge, att_edge)


def _combine_body(a_ref, wh_ref, bias_ref, out_ref):
    acc = jnp.zeros(out_ref.shape, jnp.float32)
    for h in range(H):
        acc += jnp.dot(a_ref[:, h, :], wh_ref[h], preferred_element_type=jnp.float32)
    out_ref[...] = acc * (1.0 / H) + bias_ref[...]


def _tc_combine(a_acc, W_heads, bias_mean):
    blk = 1024
    return pl.pallas_call(
        _combine_body,
        grid=(N_PAD // blk,),
        in_specs=[
            pl.BlockSpec((blk, H, D), lambda i: (i, 0, 0)),
            pl.BlockSpec((H, D, D), lambda i: (0, 0, 0)),
            pl.BlockSpec((1, D), lambda i: (0, 0)),
        ],
        out_specs=pl.BlockSpec((blk, D), lambda i: (i, 0)),
        out_shape=jax.ShapeDtypeStruct((N_PAD, D), jnp.float32),
    )(a_acc, W_heads, bias_mean)


def kernel(x, edge_index, edge_weight, W_lin, edge_emb, W_heads, att_src,
           att_dst, W_edge, att_edge, bias):
    src = edge_index[0].astype(jnp.int32)
    dst = edge_index[1].astype(jnp.int32)
    ew = edge_weight.astype(jnp.int32)

    x_lin, sa, b_sa = _tc_prep(x, W_lin, W_heads, att_src, att_dst)
    emb_pad = jnp.pad(edge_emb, ((0, EMB_PAD - EMB_ROWS), (0, 0)))
    t_tab, b_t = _tc_tscore(emb_pad, W_edge, att_edge)
    shift = b_sa[0, 0] + b_t[0, 0]

    alpha = (sa[src, :3] + sa[dst, 4:7] + t_tab[ew, :3])
    alpha = jnp.where(alpha >= 0, alpha, NEG_SLOPE * alpha)
    p = jnp.exp(alpha - shift)
    denom = jax.ops.segment_sum(p, dst, num_segments=N_NODES)
    rden = jnp.where(denom > 0, 1.0 / denom, 0.0)
    q = p * rden[dst]
    msg = q[:, :, None] * x_lin[src][:, None, :]
    a_acc = jax.ops.segment_sum(msg, dst, num_segments=N_PAD)

    bias_mean = jnp.mean(bias, axis=0, keepdims=True)
    out = _tc_combine(a_acc, W_heads, bias_mean)
    return out[:N_NODES]


# speedup vs baseline: 20.8511x; 20.8511x over previous
import functools

import jax
import jax.numpy as jnp
from jax import lax
from jax.experimental import pallas as pl
from jax.experimental.pallas import tpu as pltpu
from jax.experimental.pallas import tpu_sc as plsc

N_NODES = 10000
N_EDGES = 320000
D = 128
H = 3
EMB_ROWS = 22754
NEG_SLOPE = 0.2

EMB_PAD = 23040
N_TILE = 320
N_PAD = 32 * N_TILE

NC, NS, L = 2, 16, 16
NW = NC * NS
EPW = N_EDGES // NW
CH1 = 400
NCH1 = EPW // CH1
CH2 = 512
NCH2 = N_EDGES // CH2

_MESH = plsc.VectorSubcoreMesh(core_axis_name="c", subcore_axis_name="s")


def _wid():
    return lax.axis_index("s") * NC + lax.axis_index("c")


def _vgather(x, idx):
    dnums = lax.GatherDimensionNumbers(
        offset_dims=(), collapsed_slice_dims=(0,), start_index_map=(0,))
    return lax.gather(x, idx[:, None], dnums, slice_sizes=(1,),
                      mode=lax.GatherScatterMode.PROMISE_IN_BOUNDS)


@functools.partial(
    pl.kernel,
    out_type=jax.ShapeDtypeStruct((H * N_EDGES,), jnp.float32),
    mesh=_MESH,
    compiler_params=pltpu.CompilerParams(needs_layout_passes=False),
    scratch_types=[
        pltpu.VMEM((N_NODES * 8,), jnp.float32),
        pltpu.VMEM((CH1,), jnp.int32),
        pltpu.VMEM((CH1,), jnp.int32),
        pltpu.VMEM((H * CH1,), jnp.float32),
    ],
)
def _sc_pass1a(sa_hbm, src_hbm, dst_hbm, asd_hbm, sa_v, src_v, dst_v, ob_v):
    w = _wid()
    pltpu.sync_copy(sa_hbm, sa_v)

    def chunk(ci, _):
        base = w * EPW + ci * CH1
        pltpu.sync_copy(src_hbm.at[pl.ds(base, CH1)], src_v)
        pltpu.sync_copy(dst_hbm.at[pl.ds(base, CH1)], dst_v)

        def grp(g, _):
            s16 = src_v[pl.ds(g * L, L)]
            d16 = dst_v[pl.ds(g * L, L)]
            sb = s16 * 8
            db = d16 * 8 + 4
            for h in range(H):
                a = (plsc.load_gather(sa_v, [sb + h])
                     + plsc.load_gather(sa_v, [db + h]))
                ob_v[pl.ds(h * CH1 + g * L, L)] = a
            return 0

        lax.fori_loop(0, CH1 // L, grp, 0)
        for h in range(H):
            pltpu.sync_copy(ob_v.at[pl.ds(h * CH1, CH1)],
                            asd_hbm.at[pl.ds(h * N_EDGES + base, CH1)])
        return 0

    lax.fori_loop(0, NCH1, chunk, 0)


@functools.partial(
    pl.kernel,
    out_type=[
        jax.ShapeDtypeStruct((H * N_EDGES,), jnp.float32),
        jax.ShapeDtypeStruct((NW * N_PAD * H,), jnp.float32),
    ],
    mesh=_MESH,
    compiler_params=pltpu.CompilerParams(needs_layout_passes=False),
    scratch_types=[
        pltpu.VMEM((EMB_PAD * 4,), jnp.float32),
        pltpu.VMEM((N_PAD * H,), jnp.float32),
        pltpu.VMEM((CH1,), jnp.int32),
        pltpu.VMEM((CH1,), jnp.int32),
        pltpu.VMEM((H * CH1,), jnp.float32),
        pltpu.VMEM((H * CH1,), jnp.float32),
        pltpu.VMEM((L,), jnp.float32),
    ],
)
def _sc_pass1b(tt_hbm, ew_hbm, dst_hbm, asd_hbm, bvec_hbm, p_hbm, dpart_hbm,
               tt_v, den_v, ew_v, dst_v, ab_v, pb_v, bv_v):
    w = _wid()
    pltpu.sync_copy(tt_hbm, tt_v)
    pltpu.sync_copy(bvec_hbm, bv_v)
    zero = jnp.zeros((L,), jnp.float32)

    def z(i, _):
        den_v[pl.ds(i * L, L)] = zero
        return 0

    lax.fori_loop(0, (N_PAD * H) // L, z, 0)
    iota = jnp.arange(L, dtype=jnp.int32)
    bshift = bv_v[...]

    def chunk(ci, _):
        base = w * EPW + ci * CH1
        pltpu.sync_copy(ew_hbm.at[pl.ds(base, CH1)], ew_v)
        pltpu.sync_copy(dst_hbm.at[pl.ds(base, CH1)], dst_v)
        for h in range(H):
            pltpu.sync_copy(asd_hbm.at[pl.ds(h * N_EDGES + base, CH1)],
                            ab_v.at[pl.ds(h * CH1, CH1)])

        def grp(g, _):
            e16 = ew_v[pl.ds(g * L, L)]
            d16 = dst_v[pl.ds(g * L, L)]
            eb = e16 * 4
            ps = []
            for h in range(H):
                t = plsc.load_gather(tt_v, [eb + h])
                al = ab_v[pl.ds(h * CH1 + g * L, L)] + t
                al = jnp.where(al >= 0, al, NEG_SLOPE * al)
                p = jnp.exp(al - bshift)
                pb_v[pl.ds(h * CH1 + g * L, L)] = p
                ps.append(p)
            dsort, _unused = plsc.sort_key_val(d16, d16)
            rot = _vgather(dsort, (iota + 1) % L)
            dup = jnp.any((dsort == rot) & (iota < L - 1))
            da = d16 * H

            @pl.when(jnp.logical_not(dup))
            def _():
                for h in range(H):
                    plsc.addupdate_scatter(den_v, [da + h], ps[h])

            @pl.when(dup)
            def _():
                for j in range(L):
                    lm = iota == j
                    for h in range(H):
                        plsc.addupdate_scatter(den_v, [da + h], ps[h], mask=lm)

            return 0

        lax.fori_loop(0, CH1 // L, grp, 0)
        for h in range(H):
            pltpu.sync_copy(pb_v.at[pl.ds(h * CH1, CH1)],
                            p_hbm.at[pl.ds(h * N_EDGES + base, CH1)])
        return 0

    lax.fori_loop(0, NCH1, chunk, 0)
    pltpu.sync_copy(den_v, dpart_hbm.at[pl.ds(w * N_PAD * H, N_PAD * H)])


@functools.partial(
    pl.kernel,
    out_type=jax.ShapeDtypeStruct((N_PAD * H * D,), jnp.float32),
    mesh=_MESH,
    compiler_params=pltpu.CompilerParams(needs_layout_passes=False),
    scratch_types=[
        pltpu.VMEM((N_TILE * H * D,), jnp.float32),
        pltpu.VMEM((N_TILE * H,), jnp.float32),
        pltpu.VMEM((N_TILE * H,), jnp.float32),
        pltpu.VMEM((CH2,), jnp.int32),
        pltpu.VMEM((CH2,), jnp.int32),
        pltpu.VMEM((H * CH2,), jnp.float32),
        pltpu.VMEM((2 * L,), jnp.int32),
        pltpu.VMEM((2 * L,), jnp.int32),
        pltpu.VMEM((H * 2 * L,), jnp.float32),
        pltpu.VMEM((L, D), jnp.float32),
        pltpu.SemaphoreType.DMA,
    ],
)
def _sc_pass2(dst_hbm, src_hbm, p_hbm, dpart_hbm, xlin_hbm, a_hbm,
              acc_v, rd_v, dtmp_v, dst_v, src_v, pb_v, wls_v, wlb_v, wlq_v,
              rows_v, sem):
    w = _wid()
    lo = w * N_TILE
    zero = jnp.zeros((L,), jnp.float32)

    def zr(i, _):
        rd_v[pl.ds(i * L, L)] = zero
        return 0

    lax.fori_loop(0, (N_TILE * H) // L, zr, 0)

    def red(t, _):
        pltpu.sync_copy(dpart_hbm.at[pl.ds(t * (N_PAD * H) + lo * H, N_TILE * H)], dtmp_v)

        def addg(i, _):
            sl = pl.ds(i * L, L)
            rd_v[sl] = rd_v[sl] + dtmp_v[sl]
            return 0

        lax.fori_loop(0, (N_TILE * H) // L, addg, 0)
        return 0

    lax.fori_loop(0, NW, red, 0)

    def recip(i, _):
        sl = pl.ds(i * L, L)
        dn = rd_v[sl]
        rd_v[sl] = jnp.where(dn > 0, 1.0 / dn, 0.0)
        return 0

    lax.fori_loop(0, (N_TILE * H) // L, recip, 0)

    def z(i, _):
        acc_v[pl.ds(i * L, L)] = zero
        return 0

    lax.fori_loop(0, (N_TILE * H * D) // L, z, 0)
    iota = jnp.arange(L, dtype=jnp.int32)

    def drain():
        idx16 = wls_v[pl.ds(0, L)]
        pltpu.async_copy(xlin_hbm.at[idx16], rows_v, sem).wait()
        bvec = wlb_v[pl.ds(0, L)]
        qvecs = [wlq_v[pl.ds(h * 2 * L, L)] for h in range(H)]
        for i in range(L):
            rvecs = [rows_v[i, pl.ds(cg * L, L)] for cg in range(D // L)]
            bi = bvec[i]
            for h in range(H):
                qh = qvecs[h][i]
                for cg in range(D // L):
                    plsc.addupdate(acc_v.at[pl.ds(bi + h * D + cg * L, L)],
                                   qh * rvecs[cg])

    def shift_fwd():
        for ref in (wls_v, wlb_v):
            v = ref[pl.ds(L, L)]
            ref[pl.ds(0, L)] = v
        for h in range(H):
            v = wlq_v[pl.ds(h * 2 * L + L, L)]
            wlq_v[pl.ds(h * 2 * L, L)] = v

    def chunk(ci, wl_off):
        base = ci * CH2
        pltpu.sync_copy(dst_hbm.at[pl.ds(base, CH2)], dst_v)
        pltpu.sync_copy(src_hbm.at[pl.ds(base, CH2)], src_v)
        for h in range(H):
            pltpu.sync_copy(p_hbm.at[pl.ds(h * N_EDGES + base, CH2)],
                            pb_v.at[pl.ds(h * CH2, CH2)])

        def grp(g, off):
            d16 = dst_v[pl.ds(g * L, L)]
            own = (d16 >= lo) & (d16 < lo + N_TILE)
            cnt = jnp.sum(own.astype(jnp.int32))

            @pl.when(cnt > 0)
            def _():
                s16 = src_v[pl.ds(g * L, L)]
                dloc = jnp.where(own, d16 - lo, 0)
                plsc.store_compressed(wls_v.at[pl.ds(off, L)], s16, mask=own)
                plsc.store_compressed(wlb_v.at[pl.ds(off, L)], dloc * (H * D),
                                      mask=own)
                for h in range(H):
                    p16 = pb_v[pl.ds(h * CH2 + g * L, L)]
                    r16 = plsc.load_gather(rd_v, [dloc * H + h])
                    plsc.store_compressed(wlq_v.at[pl.ds(h * 2 * L + off, L)],
                                          p16 * r16, mask=own)

            noff = off + cnt

            @pl.when(noff >= L)
            def _():
                drain()
                shift_fwd()

            return jnp.where(noff >= L, noff - L, noff)

        return lax.fori_loop(0, CH2 // L, grp, wl_off)

    rem = lax.fori_loop(0, NCH2, chunk, jnp.int32(0))

    @pl.when(rem > 0)
    def _():
        m = iota < rem
        wls_v[pl.ds(0, L)] = jnp.where(m, wls_v[pl.ds(0, L)], iota)
        wlb_v[pl.ds(0, L)] = jnp.where(m, wlb_v[pl.ds(0, L)], 0)
        for h in range(H):
            sl = pl.ds(h * 2 * L, L)
            wlq_v[sl] = jnp.where(m, wlq_v[sl], 0.0)
        drain()

    pltpu.sync_copy(acc_v, a_hbm.at[pl.ds(w * N_TILE * H * D, N_TILE * H * D)])


def _prep_body(x_ref, wlin_ref, wh_ref, asrc_ref, adst_ref,
               xlin_ref, sa_ref, bmax_ref):
    i = pl.program_id(0)
    xlin = jnp.dot(x_ref[...], wlin_ref[...], preferred_element_type=jnp.float32)
    xlin_ref[...] = xlin
    cols = []
    for h in range(H):
        cols.append(jnp.dot(wh_ref[h], asrc_ref[h], preferred_element_type=jnp.float32))
    cols.append(jnp.zeros((D,), jnp.float32))
    for h in range(H):
        cols.append(jnp.dot(wh_ref[h], adst_ref[h], preferred_element_type=jnp.float32))
    cols.append(jnp.zeros((D,), jnp.float32))
    u8 = jnp.stack(cols, axis=1)
    sa = jnp.dot(xlin, u8, preferred_element_type=jnp.float32)
    sa_ref[...] = sa
    blk_max = (jnp.max(sa[:, :3]) + jnp.max(sa[:, 4:7])).reshape(1, 1)

    @pl.when(i == 0)
    def _():
        bmax_ref[...] = blk_max

    @pl.when(i > 0)
    def _():
        bmax_ref[...] = jnp.maximum(bmax_ref[...], blk_max)


def _tc_prep(x, W_lin, W_heads, att_src, att_dst):
    blk = 1000
    return pl.pallas_call(
        _prep_body,
        grid=(N_NODES // blk,),
        in_specs=[
            pl.BlockSpec((blk, D), lambda i: (i, 0)),
            pl.BlockSpec((D, D), lambda i: (0, 0)),
            pl.BlockSpec((H, D, D), lambda i: (0, 0, 0)),
            pl.BlockSpec((H, D), lambda i: (0, 0)),
            pl.BlockSpec((H, D), lambda i: (0, 0)),
        ],
        out_specs=[
            pl.BlockSpec((blk, D), lambda i: (i, 0)),
            pl.BlockSpec((blk, 8), lambda i: (i, 0)),
            pl.BlockSpec((1, 1), lambda i: (0, 0)),
        ],
        out_shape=[
            jax.ShapeDtypeStruct((N_NODES, D), jnp.float32),
            jax.ShapeDtypeStruct((N_NODES, 8), jnp.float32),
            jax.ShapeDtypeStruct((1, 1), jnp.float32),
        ],
    )(x, W_lin, W_heads, att_src, att_dst)


def _tscore_body(emb_ref, we_ref, ae_ref, t_ref, tmax_ref):
    i = pl.program_id(0)
    cols = []
    for h in range(H):
        cols.append(jnp.dot(we_ref[h], ae_ref[h], preferred_element_type=jnp.float32))
    cols.append(jnp.zeros((D,), jnp.float32))
    v4 = jnp.stack(cols, axis=1)
    t = jnp.dot(emb_ref[...], v4, preferred_element_type=jnp.float32)
    t_ref[...] = t
    blk_max = jnp.max(t[:, :3]).reshape(1, 1)

    @pl.when(i == 0)
    def _():
        tmax_ref[...] = blk_max

    @pl.when(i > 0)
    def _():
        tmax_ref[...] = jnp.maximum(tmax_ref[...], blk_max)


def _tc_tscore(emb_pad, W_edge, att_edge):
    blk = 512
    return pl.pallas_call(
        _tscore_body,
        grid=(EMB_PAD // blk,),
        in_specs=[
            pl.BlockSpec((blk, D), lambda i: (i, 0)),
            pl.BlockSpec((H, D, D), lambda i: (0, 0, 0)),
            pl.BlockSpec((H, D), lambda i: (0, 0)),
        ],
        out_specs=[
            pl.BlockSpec((blk, 4), lambda i: (i, 0)),
            pl.BlockSpec((1, 1), lambda i: (0, 0)),
        ],
        out_shape=[
            jax.ShapeDtypeStruct((EMB_PAD, 4), jnp.float32),
            jax.ShapeDtypeStruct((1, 1), jnp.float32),
        ],
    )(emb_pad, W_edge, att_edge)


def _combine_body(a_ref, wh_ref, bias_ref, out_ref):
    acc = jnp.zeros(out_ref.shape, jnp.float32)
    for h in range(H):
        acc += jnp.dot(a_ref[:, h, :], wh_ref[h], preferred_element_type=jnp.float32)
    out_ref[...] = acc * (1.0 / H) + bias_ref[...]


def _tc_combine(a_acc, W_heads, bias_mean):
    blk = 1024
    return pl.pallas_call(
        _combine_body,
        grid=(N_PAD // blk,),
        in_specs=[
            pl.BlockSpec((blk, H, D), lambda i: (i, 0, 0)),
            pl.BlockSpec((H, D, D), lambda i: (0, 0, 0)),
            pl.BlockSpec((1, D), lambda i: (0, 0)),
        ],
        out_specs=pl.BlockSpec((blk, D), lambda i: (i, 0)),
        out_shape=jax.ShapeDtypeStruct((N_PAD, D), jnp.float32),
    )(a_acc, W_heads, bias_mean)


def kernel(x, edge_index, edge_weight, W_lin, edge_emb, W_heads, att_src,
           att_dst, W_edge, att_edge, bias):
    src = edge_index[0].astype(jnp.int32)
    dst = edge_index[1].astype(jnp.int32)
    ew = edge_weight.astype(jnp.int32)

    x_lin, sa, b_sa = _tc_prep(x, W_lin, W_heads, att_src, att_dst)
    emb_pad = jnp.pad(edge_emb, ((0, EMB_PAD - EMB_ROWS), (0, 0)))
    t_tab, b_t = _tc_tscore(emb_pad, W_edge, att_edge)
    bvec = jnp.full((L,), b_sa[0, 0] + b_t[0, 0], jnp.float32)

    asd = _sc_pass1a(sa.reshape(-1), src, dst)
    p_flat, dpart = _sc_pass1b(t_tab.reshape(-1), ew, dst, asd, bvec)
    a_flat = _sc_pass2(dst, src, p_flat, dpart, x_lin)
    a_acc = a_flat.reshape(N_PAD, H, D)

    bias_mean = jnp.mean(bias, axis=0, keepdims=True)
    out = _tc_combine(a_acc, W_heads, bias_mean)
    return out[:N_NODES]


# speedup vs baseline: 47.5488x; 2.2804x over previous
import functools

import jax
import jax.numpy as jnp
from jax import lax
from jax.experimental import pallas as pl
from jax.experimental.pallas import tpu as pltpu
from jax.experimental.pallas import tpu_sc as plsc

N_NODES = 10000
N_EDGES = 320000
D = 128
H = 3
EMB_ROWS = 22754
NEG_SLOPE = 0.2

EMB_PAD = 23040
N_TILE = 320
N_PAD = 32 * N_TILE

NC, NS, L = 2, 16, 16
NW = NC * NS
EPW = N_EDGES // NW
CH1 = 400
NCH1 = EPW // CH1
CH2 = 400
NCH2 = N_EDGES // CH2

_MESH = plsc.VectorSubcoreMesh(core_axis_name="c", subcore_axis_name="s")


def _wid():
    return lax.axis_index("s") * NC + lax.axis_index("c")


def _vgather(x, idx):
    dnums = lax.GatherDimensionNumbers(
        offset_dims=(), collapsed_slice_dims=(0,), start_index_map=(0,))
    return lax.gather(x, idx[:, None], dnums, slice_sizes=(1,),
                      mode=lax.GatherScatterMode.PROMISE_IN_BOUNDS)


@functools.partial(
    pl.kernel,
    out_type=jax.ShapeDtypeStruct((H * N_EDGES,), jnp.float32),
    mesh=_MESH,
    compiler_params=pltpu.CompilerParams(needs_layout_passes=False),
    scratch_types=[
        pltpu.VMEM((N_NODES * 8,), jnp.float32),
        pltpu.VMEM((CH1,), jnp.int32),
        pltpu.VMEM((CH1,), jnp.int32),
        pltpu.VMEM((H * CH1,), jnp.float32),
    ],
)
def _sc_pass1a(sa_hbm, src_hbm, dst_hbm, asd_hbm, sa_v, src_v, dst_v, ob_v):
    w = _wid()
    pltpu.sync_copy(sa_hbm, sa_v)

    def chunk(ci, _):
        base = w * EPW + ci * CH1
        pltpu.sync_copy(src_hbm.at[pl.ds(base, CH1)], src_v)
        pltpu.sync_copy(dst_hbm.at[pl.ds(base, CH1)], dst_v)

        def grp(g, _):
            s16 = src_v[pl.ds(g * L, L)]
            d16 = dst_v[pl.ds(g * L, L)]
            sb = s16 * 8
            db = d16 * 8 + 4
            for h in range(H):
                a = (plsc.load_gather(sa_v, [sb + h])
                     + plsc.load_gather(sa_v, [db + h]))
                ob_v[pl.ds(h * CH1 + g * L, L)] = a
            return 0

        lax.fori_loop(0, CH1 // L, grp, 0)
        for h in range(H):
            pltpu.sync_copy(ob_v.at[pl.ds(h * CH1, CH1)],
                            asd_hbm.at[pl.ds(h * N_EDGES + base, CH1)])
        return 0

    lax.fori_loop(0, NCH1, chunk, 0)


@functools.partial(
    pl.kernel,
    out_type=[
        jax.ShapeDtypeStruct((H * N_EDGES,), jnp.float32),
        jax.ShapeDtypeStruct((NW * N_PAD * H,), jnp.float32),
    ],
    mesh=_MESH,
    compiler_params=pltpu.CompilerParams(needs_layout_passes=False),
    scratch_types=[
        pltpu.VMEM((EMB_PAD * 4,), jnp.float32),
        pltpu.VMEM((N_PAD * H,), jnp.float32),
        pltpu.VMEM((CH1,), jnp.int32),
        pltpu.VMEM((CH1,), jnp.int32),
        pltpu.VMEM((H * CH1,), jnp.float32),
        pltpu.VMEM((H * CH1,), jnp.float32),
        pltpu.VMEM((L,), jnp.float32),
    ],
)
def _sc_pass1b(tt_hbm, ew_hbm, dst_hbm, asd_hbm, bvec_hbm, p_hbm, dpart_hbm,
               tt_v, den_v, ew_v, dst_v, ab_v, pb_v, bv_v):
    w = _wid()
    pltpu.sync_copy(tt_hbm, tt_v)
    pltpu.sync_copy(bvec_hbm, bv_v)
    zero = jnp.zeros((L,), jnp.float32)

    def z(i, _):
        den_v[pl.ds(i * L, L)] = zero
        return 0

    lax.fori_loop(0, (N_PAD * H) // L, z, 0)
    iota = jnp.arange(L, dtype=jnp.int32)
    bshift = bv_v[...]

    def chunk(ci, _):
        base = w * EPW + ci * CH1
        pltpu.sync_copy(ew_hbm.at[pl.ds(base, CH1)], ew_v)
        pltpu.sync_copy(dst_hbm.at[pl.ds(base, CH1)], dst_v)
        for h in range(H):
            pltpu.sync_copy(asd_hbm.at[pl.ds(h * N_EDGES + base, CH1)],
                            ab_v.at[pl.ds(h * CH1, CH1)])

        def grp(g, _):
            e16 = ew_v[pl.ds(g * L, L)]
            d16 = dst_v[pl.ds(g * L, L)]
            eb = e16 * 4
            ps = []
            for h in range(H):
                t = plsc.load_gather(tt_v, [eb + h])
                al = ab_v[pl.ds(h * CH1 + g * L, L)] + t
                al = jnp.where(al >= 0, al, NEG_SLOPE * al)
                p = jnp.exp(al - bshift)
                pb_v[pl.ds(h * CH1 + g * L, L)] = p
                ps.append(p)
            dsort, _unused = plsc.sort_key_val(d16, d16)
            rot = _vgather(dsort, (iota + 1) % L)
            dup = jnp.any((dsort == rot) & (iota < L - 1))
            da = d16 * H

            @pl.when(jnp.logical_not(dup))
            def _():
                for h in range(H):
                    plsc.addupdate_scatter(den_v, [da + h], ps[h])

            @pl.when(dup)
            def _():
                for j in range(L):
                    lm = iota == j
                    for h in range(H):
                        plsc.addupdate_scatter(den_v, [da + h], ps[h], mask=lm)

            return 0

        lax.fori_loop(0, CH1 // L, grp, 0)
        cj = w * NCH1 + ci
        for h in range(H):
            pltpu.sync_copy(pb_v.at[pl.ds(h * CH1, CH1)],
                            p_hbm.at[pl.ds(cj * H * CH1 + h * CH1, CH1)])
        return 0

    lax.fori_loop(0, NCH1, chunk, 0)
    pltpu.sync_copy(den_v, dpart_hbm.at[pl.ds(w * N_PAD * H, N_PAD * H)])


@functools.partial(
    pl.kernel,
    out_type=jax.ShapeDtypeStruct((N_PAD * H * D,), jnp.float32),
    mesh=_MESH,
    compiler_params=pltpu.CompilerParams(needs_layout_passes=False),
    scratch_types=[
        pltpu.VMEM((N_TILE * H * D,), jnp.float32),
        pltpu.VMEM((N_TILE * H,), jnp.float32),
        pltpu.VMEM((2 * 2 * CH2,), jnp.int32),
        pltpu.VMEM((2 * H * CH2,), jnp.float32),
        pltpu.VMEM((2 * L,), jnp.int32),
        pltpu.VMEM((2 * L,), jnp.int32),
        pltpu.VMEM((H * 2 * L,), jnp.float32),
        pltpu.VMEM((L,), jnp.int32),
        pltpu.VMEM((H * L,), jnp.float32),
        pltpu.VMEM((L, D), jnp.float32),
        pltpu.SemaphoreType.DMA,
        pltpu.SemaphoreType.DMA,
    ],
)
def _sc_pass2(dsrec_hbm, p_hbm, dpart_hbm, xlin_hbm, a_hbm,
              acc_v, rd_v, rec_v, prec_v, wls_v, wlb_v, wlq_v, pb2_v, pq2_v,
              rows_v, sem_s, sem_r):
    w = _wid()
    lo = w * N_TILE
    zero = jnp.zeros((L,), jnp.float32)

    def zr(i, _):
        rd_v[pl.ds(i * L, L)] = zero
        return 0

    lax.fori_loop(0, (N_TILE * H) // L, zr, 0)

    def red(t, _):
        pltpu.sync_copy(
            dpart_hbm.at[pl.ds(t * (N_PAD * H) + lo * H, N_TILE * H)],
            prec_v.at[pl.ds(0, N_TILE * H)])

        def addg(i, _):
            sl = pl.ds(i * L, L)
            rd_v[sl] = rd_v[sl] + prec_v[sl]
            return 0

        lax.fori_loop(0, (N_TILE * H) // L, addg, 0)
        return 0

    lax.fori_loop(0, NW, red, 0)

    def recip(i, _):
        sl = pl.ds(i * L, L)
        dn = rd_v[sl]
        rd_v[sl] = jnp.where(dn > 0, 1.0 / dn, 0.0)
        return 0

    lax.fori_loop(0, (N_TILE * H) // L, recip, 0)

    def z(i, _):
        acc_v[pl.ds(i * L, L)] = zero
        return 0

    lax.fori_loop(0, (N_TILE * H * D) // L, z, 0)
    iota = jnp.arange(L, dtype=jnp.int32)

    def stage(ci, par):
        pltpu.async_copy(dsrec_hbm.at[pl.ds(ci * 2 * CH2, 2 * CH2)],
                         rec_v.at[pl.ds(par * 2 * CH2, 2 * CH2)], sem_s)
        pltpu.async_copy(p_hbm.at[pl.ds(ci * H * CH2, H * CH2)],
                         prec_v.at[pl.ds(par * H * CH2, H * CH2)], sem_s)

    def wait_stage(ci, par):
        pltpu.make_async_copy(
            dsrec_hbm.at[pl.ds(ci * 2 * CH2, 2 * CH2)],
            rec_v.at[pl.ds(par * 2 * CH2, 2 * CH2)], sem_s).wait()
        pltpu.make_async_copy(
            p_hbm.at[pl.ds(ci * H * CH2, H * CH2)],
            prec_v.at[pl.ds(par * H * CH2, H * CH2)], sem_s).wait()

    def fire_pending():
        idx16 = wls_v[pl.ds(0, L)]
        pltpu.async_copy(xlin_hbm.at[idx16], rows_v, sem_r)
        pb2_v[pl.ds(0, L)] = wlb_v[pl.ds(0, L)]
        for h in range(H):
            pq2_v[pl.ds(h * L, L)] = wlq_v[pl.ds(h * 2 * L, L)]

    def acc_pending():
        pltpu.make_async_copy(xlin_hbm.at[pl.ds(0, L)], rows_v, sem_r).wait()
        bvec = pb2_v[pl.ds(0, L)]
        qvecs = [pq2_v[pl.ds(h * L, L)] for h in range(H)]
        for i in range(L):
            rvecs = [rows_v[i, pl.ds(cg * L, L)] for cg in range(D // L)]
            bi = bvec[i]
            for h in range(H):
                qh = qvecs[h][i]
                for cg in range(D // L):
                    plsc.addupdate(acc_v.at[pl.ds(bi + h * D + cg * L, L)],
                                   qh * rvecs[cg])

    def shift_fwd():
        for ref in (wls_v, wlb_v):
            v = ref[pl.ds(L, L)]
            ref[pl.ds(0, L)] = v
        for h in range(H):
            v = wlq_v[pl.ds(h * 2 * L + L, L)]
            wlq_v[pl.ds(h * 2 * L, L)] = v

    stage(0, 0)

    def chunk(ci, carry):
        off0, pend0 = carry
        par = lax.rem(ci, 2)
        wait_stage(ci, par)

        @pl.when(ci + 1 < NCH2)
        def _():
            stage(ci + 1, 1 - par)

        def grp(g, c):
            off, pend = c
            d16 = rec_v[pl.ds(par * 2 * CH2 + g * L, L)]
            own = (d16 >= lo) & (d16 < lo + N_TILE)
            cnt = jnp.sum(own.astype(jnp.int32))

            @pl.when(cnt > 0)
            def _():
                s16 = rec_v[pl.ds(par * 2 * CH2 + CH2 + g * L, L)]
                dloc = jnp.where(own, d16 - lo, 0)
                plsc.store_compressed(wls_v.at[pl.ds(off, L)], s16, mask=own)
                plsc.store_compressed(wlb_v.at[pl.ds(off, L)], dloc * (H * D),
                                      mask=own)
                for h in range(H):
                    p16 = prec_v[pl.ds(par * H * CH2 + h * CH2 + g * L, L)]
                    r16 = plsc.load_gather(rd_v, [dloc * H + h])
                    plsc.store_compressed(wlq_v.at[pl.ds(h * 2 * L + off, L)],
                                          p16 * r16, mask=own)

            noff = off + cnt

            @pl.when(noff >= L)
            def _():
                @pl.when(pend == 1)
                def _():
                    acc_pending()

                fire_pending()
                shift_fwd()

            return (jnp.where(noff >= L, noff - L, noff),
                    jnp.where(noff >= L, 1, pend))

        return lax.fori_loop(0, CH2 // L, grp, (off0, pend0))

    rem, pend = lax.fori_loop(0, NCH2, chunk, (jnp.int32(0), jnp.int32(0)))

    @pl.when(pend == 1)
    def _():
        acc_pending()

    @pl.when(rem > 0)
    def _():
        m = iota < rem
        wls_v[pl.ds(0, L)] = jnp.where(m, wls_v[pl.ds(0, L)], iota)
        wlb_v[pl.ds(0, L)] = jnp.where(m, wlb_v[pl.ds(0, L)], 0)
        for h in range(H):
            sl = pl.ds(h * 2 * L, L)
            wlq_v[sl] = jnp.where(m, wlq_v[sl], 0.0)
        fire_pending()
        acc_pending()

    pltpu.sync_copy(acc_v, a_hbm.at[pl.ds(w * N_TILE * H * D, N_TILE * H * D)])


def _prep_body(x_ref, wlin_ref, wh_ref, asrc_ref, adst_ref,
               xlin_ref, sa_ref, bmax_ref):
    i = pl.program_id(0)
    xlin = jnp.dot(x_ref[...], wlin_ref[...], preferred_element_type=jnp.float32)
    xlin_ref[...] = xlin
    cols = []
    for h in range(H):
        cols.append(jnp.dot(wh_ref[h], asrc_ref[h], preferred_element_type=jnp.float32))
    cols.append(jnp.zeros((D,), jnp.float32))
    for h in range(H):
        cols.append(jnp.dot(wh_ref[h], adst_ref[h], preferred_element_type=jnp.float32))
    cols.append(jnp.zeros((D,), jnp.float32))
    u8 = jnp.stack(cols, axis=1)
    sa = jnp.dot(xlin, u8, preferred_element_type=jnp.float32)
    sa_ref[...] = sa
    blk_max = (jnp.max(sa[:, :3]) + jnp.max(sa[:, 4:7])).reshape(1, 1)

    @pl.when(i == 0)
    def _():
        bmax_ref[...] = blk_max

    @pl.when(i > 0)
    def _():
        bmax_ref[...] = jnp.maximum(bmax_ref[...], blk_max)


def _tc_prep(x, W_lin, W_heads, att_src, att_dst):
    blk = 1000
    return pl.pallas_call(
        _prep_body,
        grid=(N_NODES // blk,),
        in_specs=[
            pl.BlockSpec((blk, D), lambda i: (i, 0)),
            pl.BlockSpec((D, D), lambda i: (0, 0)),
            pl.BlockSpec((H, D, D), lambda i: (0, 0, 0)),
            pl.BlockSpec((H, D), lambda i: (0, 0)),
            pl.BlockSpec((H, D), lambda i: (0, 0)),
        ],
        out_specs=[
            pl.BlockSpec((blk, D), lambda i: (i, 0)),
            pl.BlockSpec((blk, 8), lambda i: (i, 0)),
            pl.BlockSpec((1, 1), lambda i: (0, 0)),
        ],
        out_shape=[
            jax.ShapeDtypeStruct((N_NODES, D), jnp.float32),
            jax.ShapeDtypeStruct((N_NODES, 8), jnp.float32),
            jax.ShapeDtypeStruct((1, 1), jnp.float32),
        ],
    )(x, W_lin, W_heads, att_src, att_dst)


def _tscore_body(emb_ref, we_ref, ae_ref, t_ref, tmax_ref):
    i = pl.program_id(0)
    cols = []
    for h in range(H):
        cols.append(jnp.dot(we_ref[h], ae_ref[h], preferred_element_type=jnp.float32))
    cols.append(jnp.zeros((D,), jnp.float32))
    v4 = jnp.stack(cols, axis=1)
    t = jnp.dot(emb_ref[...], v4, preferred_element_type=jnp.float32)
    t_ref[...] = t
    blk_max = jnp.max(t[:, :3]).reshape(1, 1)

    @pl.when(i == 0)
    def _():
        tmax_ref[...] = blk_max

    @pl.when(i > 0)
    def _():
        tmax_ref[...] = jnp.maximum(tmax_ref[...], blk_max)


def _tc_tscore(emb_pad, W_edge, att_edge):
    blk = 512
    return pl.pallas_call(
        _tscore_body,
        grid=(EMB_PAD // blk,),
        in_specs=[
            pl.BlockSpec((blk, D), lambda i: (i, 0)),
            pl.BlockSpec((H, D, D), lambda i: (0, 0, 0)),
            pl.BlockSpec((H, D), lambda i: (0, 0)),
        ],
        out_specs=[
            pl.BlockSpec((blk, 4), lambda i: (i, 0)),
            pl.BlockSpec((1, 1), lambda i: (0, 0)),
        ],
        out_shape=[
            jax.ShapeDtypeStruct((EMB_PAD, 4), jnp.float32),
            jax.ShapeDtypeStruct((1, 1), jnp.float32),
        ],
    )(emb_pad, W_edge, att_edge)


def _combine_body(a_ref, wh_ref, bias_ref, out_ref):
    acc = jnp.zeros(out_ref.shape, jnp.float32)
    for h in range(H):
        acc += jnp.dot(a_ref[:, h, :], wh_ref[h], preferred_element_type=jnp.float32)
    out_ref[...] = acc * (1.0 / H) + bias_ref[...]


def _tc_combine(a_acc, W_heads, bias_mean):
    blk = 1024
    return pl.pallas_call(
        _combine_body,
        grid=(N_PAD // blk,),
        in_specs=[
            pl.BlockSpec((blk, H, D), lambda i: (i, 0, 0)),
            pl.BlockSpec((H, D, D), lambda i: (0, 0, 0)),
            pl.BlockSpec((1, D), lambda i: (0, 0)),
        ],
        out_specs=pl.BlockSpec((blk, D), lambda i: (i, 0)),
        out_shape=jax.ShapeDtypeStruct((N_PAD, D), jnp.float32),
    )(a_acc, W_heads, bias_mean)


def kernel(x, edge_index, edge_weight, W_lin, edge_emb, W_heads, att_src,
           att_dst, W_edge, att_edge, bias):
    src = edge_index[0].astype(jnp.int32)
    dst = edge_index[1].astype(jnp.int32)
    ew = edge_weight.astype(jnp.int32)

    x_lin, sa, b_sa = _tc_prep(x, W_lin, W_heads, att_src, att_dst)
    emb_pad = jnp.pad(edge_emb, ((0, EMB_PAD - EMB_ROWS), (0, 0)))
    t_tab, b_t = _tc_tscore(emb_pad, W_edge, att_edge)
    bvec = jnp.full((L,), b_sa[0, 0] + b_t[0, 0], jnp.float32)

    asd = _sc_pass1a(sa.reshape(-1), src, dst)
    p_flat, dpart = _sc_pass1b(t_tab.reshape(-1), ew, dst, asd, bvec)
    dsrec = jnp.concatenate([dst.reshape(NCH2, 1, CH2),
                             src.reshape(NCH2, 1, CH2)], axis=1).reshape(-1)
    a_flat = _sc_pass2(dsrec, p_flat, dpart, x_lin)
    a_acc = a_flat.reshape(N_PAD, H, D)

    bias_mean = jnp.mean(bias, axis=0, keepdims=True)
    out = _tc_combine(a_acc, W_heads, bias_mean)
    return out[:N_NODES]


# speedup vs baseline: 67.5608x; 1.4209x over previous
import functools

import jax
import jax.numpy as jnp
from jax import lax
from jax.experimental import pallas as pl
from jax.experimental.pallas import tpu as pltpu
from jax.experimental.pallas import tpu_sc as plsc

N_NODES = 10000
N_EDGES = 320000
D = 128
H = 3
EMB_ROWS = 22754
NEG_SLOPE = 0.2

EMB_PAD = 23040
N_TILE = 320
N_PAD = 32 * N_TILE

NC, NS, L = 2, 16, 16
NW = NC * NS
EPW = N_EDGES // NW
CH1 = 400
NCH1 = EPW // CH1
CH2 = 400
NCH2 = N_EDGES // CH2
CAP = 448
BKT = 5 * CAP
TREG = NW * BKT

_MESH = plsc.VectorSubcoreMesh(core_axis_name="c", subcore_axis_name="s")


def _wid():
    return lax.axis_index("s") * NC + lax.axis_index("c")


def _vgather(x, idx):
    dnums = lax.GatherDimensionNumbers(
        offset_dims=(), collapsed_slice_dims=(0,), start_index_map=(0,))
    return lax.gather(x, idx[:, None], dnums, slice_sizes=(1,),
                      mode=lax.GatherScatterMode.PROMISE_IN_BOUNDS)


@functools.partial(
    pl.kernel,
    out_type=jax.ShapeDtypeStruct((H * N_EDGES,), jnp.float32),
    mesh=_MESH,
    compiler_params=pltpu.CompilerParams(needs_layout_passes=False),
    scratch_types=[
        pltpu.VMEM((N_NODES * 8,), jnp.float32),
        pltpu.VMEM((CH1,), jnp.int32),
        pltpu.VMEM((CH1,), jnp.int32),
        pltpu.VMEM((H * CH1,), jnp.float32),
    ],
)
def _sc_pass1a(sa_hbm, src_hbm, dst_hbm, asd_hbm, sa_v, src_v, dst_v, ob_v):
    w = _wid()
    pltpu.sync_copy(sa_hbm, sa_v)

    def chunk(ci, _):
        base = w * EPW + ci * CH1
        pltpu.sync_copy(src_hbm.at[pl.ds(base, CH1)], src_v)
        pltpu.sync_copy(dst_hbm.at[pl.ds(base, CH1)], dst_v)

        def grp(g, _):
            s16 = src_v[pl.ds(g * L, L)]
            d16 = dst_v[pl.ds(g * L, L)]
            sb = s16 * 8
            db = d16 * 8 + 4
            for h in range(H):
                a = (plsc.load_gather(sa_v, [sb + h])
                     + plsc.load_gather(sa_v, [db + h]))
                ob_v[pl.ds(h * CH1 + g * L, L)] = a
            return 0

        lax.fori_loop(0, CH1 // L, grp, 0)
        for h in range(H):
            pltpu.sync_copy(ob_v.at[pl.ds(h * CH1, CH1)],
                            asd_hbm.at[pl.ds(h * N_EDGES + base, CH1)])
        return 0

    lax.fori_loop(0, NCH1, chunk, 0)


@functools.partial(
    pl.kernel,
    out_type=[
        jax.ShapeDtypeStruct((H * N_EDGES,), jnp.float32),
        jax.ShapeDtypeStruct((NW * N_PAD * H,), jnp.float32),
    ],
    mesh=_MESH,
    compiler_params=pltpu.CompilerParams(needs_layout_passes=False),
    scratch_types=[
        pltpu.VMEM((EMB_PAD * 4,), jnp.float32),
        pltpu.VMEM((N_PAD * H,), jnp.float32),
        pltpu.VMEM((CH1,), jnp.int32),
        pltpu.VMEM((CH1,), jnp.int32),
        pltpu.VMEM((H * CH1,), jnp.float32),
        pltpu.VMEM((H * CH1,), jnp.float32),
        pltpu.VMEM((L,), jnp.float32),
    ],
)
def _sc_pass1b(tt_hbm, ew_hbm, dst_hbm, asd_hbm, bvec_hbm, p_hbm, dpart_hbm,
               tt_v, den_v, ew_v, dst_v, ab_v, pb_v, bv_v):
    w = _wid()
    pltpu.sync_copy(tt_hbm, tt_v)
    pltpu.sync_copy(bvec_hbm, bv_v)
    zero = jnp.zeros((L,), jnp.float32)

    def z(i, _):
        den_v[pl.ds(i * L, L)] = zero
        return 0

    lax.fori_loop(0, (N_PAD * H) // L, z, 0)
    iota = jnp.arange(L, dtype=jnp.int32)
    bshift = bv_v[...]

    def chunk(ci, _):
        base = w * EPW + ci * CH1
        pltpu.sync_copy(ew_hbm.at[pl.ds(base, CH1)], ew_v)
        pltpu.sync_copy(dst_hbm.at[pl.ds(base, CH1)], dst_v)
        for h in range(H):
            pltpu.sync_copy(asd_hbm.at[pl.ds(h * N_EDGES + base, CH1)],
                            ab_v.at[pl.ds(h * CH1, CH1)])

        def grp(g, _):
            e16 = ew_v[pl.ds(g * L, L)]
            d16 = dst_v[pl.ds(g * L, L)]
            eb = e16 * 4
            ps = []
            for h in range(H):
                t = plsc.load_gather(tt_v, [eb + h])
                al = ab_v[pl.ds(h * CH1 + g * L, L)] + t
                al = jnp.where(al >= 0, al, NEG_SLOPE * al)
                p = jnp.exp(al - bshift)
                pb_v[pl.ds(h * CH1 + g * L, L)] = p
                ps.append(p)
            dsort, _unused = plsc.sort_key_val(d16, d16)
            rot = _vgather(dsort, (iota + 1) % L)
            dup = jnp.any((dsort == rot) & (iota < L - 1))
            da = d16 * H

            @pl.when(jnp.logical_not(dup))
            def _():
                for h in range(H):
                    plsc.addupdate_scatter(den_v, [da + h], ps[h])

            @pl.when(dup)
            def _():
                for j in range(L):
                    lm = iota == j
                    for h in range(H):
                        plsc.addupdate_scatter(den_v, [da + h], ps[h], mask=lm)

            return 0

        lax.fori_loop(0, CH1 // L, grp, 0)
        cj = w * NCH1 + ci
        for h in range(H):
            pltpu.sync_copy(pb_v.at[pl.ds(h * CH1, CH1)],
                            p_hbm.at[pl.ds(cj * H * CH1 + h * CH1, CH1)])
        return 0

    lax.fori_loop(0, NCH1, chunk, 0)
    pltpu.sync_copy(den_v, dpart_hbm.at[pl.ds(w * N_PAD * H, N_PAD * H)])


@functools.partial(
    pl.kernel,
    out_type=[
        jax.ShapeDtypeStruct((NW * TREG,), jnp.int32),
        jax.ShapeDtypeStruct((NW * NW + L,), jnp.int32),
    ],
    mesh=_MESH,
    compiler_params=pltpu.CompilerParams(needs_layout_passes=False),
    scratch_types=[
        pltpu.VMEM((TREG,), jnp.int32),
        pltpu.VMEM((NW,), jnp.int32),
        pltpu.VMEM((2 * 2 * CH2,), jnp.int32),
        pltpu.VMEM((2 * H * CH2,), jnp.float32),
        pltpu.SemaphoreType.DMA,
    ],
)
def _sc_pass1c(dsrec_hbm, p_hbm, binned_hbm, cnts_hbm,
               bkt_v, cnt_v, rec_v, prec_v, sem_s):
    w = _wid()
    zi = jnp.zeros((L,), jnp.int32)
    cnt_v[pl.ds(0, L)] = zi
    cnt_v[pl.ds(L, L)] = zi
    iota = jnp.arange(L, dtype=jnp.int32)

    def zb(i, _):
        bkt_v[pl.ds(i * L, L)] = zi
        return 0

    lax.fori_loop(0, TREG // L, zb, 0)

    def stage(ci, par):
        cj = w * NCH1 + ci
        pltpu.async_copy(dsrec_hbm.at[pl.ds(cj * 2 * CH2, 2 * CH2)],
                         rec_v.at[pl.ds(par * 2 * CH2, 2 * CH2)], sem_s)
        pltpu.async_copy(p_hbm.at[pl.ds(cj * H * CH2, H * CH2)],
                         prec_v.at[pl.ds(par * H * CH2, H * CH2)], sem_s)

    def wait_stage(ci, par):
        cj = w * NCH1 + ci
        pltpu.make_async_copy(
            dsrec_hbm.at[pl.ds(cj * 2 * CH2, 2 * CH2)],
            rec_v.at[pl.ds(par * 2 * CH2, 2 * CH2)], sem_s).wait()
        pltpu.make_async_copy(
            p_hbm.at[pl.ds(cj * H * CH2, H * CH2)],
            prec_v.at[pl.ds(par * H * CH2, H * CH2)], sem_s).wait()

    stage(0, 0)

    def chunk(ci, cnts):
        par = lax.rem(ci, 2)
        wait_stage(ci, par)

        @pl.when(ci + 1 < NCH1)
        def _():
            stage(ci + 1, 1 - par)

        def grp(g, cc):
            c_lo, c_hi = cc
            d16 = rec_v[pl.ds(par * 2 * CH2 + g * L, L)]
            s16 = rec_v[pl.ds(par * 2 * CH2 + CH2 + g * L, L)]
            owner = jnp.right_shift(d16 * 6554, 21)
            dloc = d16 - owner * N_TILE
            dbase = dloc * (H * D)
            p16s = [prec_v[pl.ds(par * H * CH2 + h * CH2 + g * L, L)]
                    for h in range(H)]
            for j in range(L):
                mj = iota == j
                oj = owner[j]
                sel = oj < L
                oid = jnp.bitwise_and(oj, L - 1)
                csel = jnp.where(sel, c_lo, c_hi)
                cj = _vgather(csel, jnp.full((L,), oid, jnp.int32))
                pos = jnp.minimum(cj, CAP - 1)
                base = oj * BKT + pos
                plsc.store_scatter(bkt_v, [base], s16, mask=mj)
                plsc.store_scatter(bkt_v, [base + CAP], dbase, mask=mj)
                for h in range(H):
                    plsc.store_scatter(bkt_v, [base + (2 + h) * CAP],
                                       plsc.bitcast(p16s[h], jnp.int32),
                                       mask=mj)
                hit = iota == oid
                c_lo = c_lo + jnp.where(hit & sel, 1, 0)
                c_hi = c_hi + jnp.where(hit & jnp.logical_not(sel), 1, 0)
            return (c_lo, c_hi)

        return lax.fori_loop(0, CH2 // L, grp, cnts)

    zi32 = jnp.zeros((L,), jnp.int32)
    c_lo, c_hi = lax.fori_loop(0, NCH1, chunk, (zi32, zi32))
    cnt_v[pl.ds(0, L)] = c_lo
    cnt_v[pl.ds(L, L)] = c_hi
    pltpu.sync_copy(bkt_v, binned_hbm.at[pl.ds(w * TREG, TREG)])
    pltpu.sync_copy(cnt_v, cnts_hbm.at[pl.ds(w * NW, NW)])


@functools.partial(
    pl.kernel,
    out_type=jax.ShapeDtypeStruct((N_PAD * H * D,), jnp.float32),
    mesh=_MESH,
    compiler_params=pltpu.CompilerParams(needs_layout_passes=False),
    scratch_types=[
        pltpu.VMEM((N_TILE * H * D,), jnp.float32),
        pltpu.VMEM((N_TILE * H,), jnp.float32),
        pltpu.VMEM((N_TILE * H,), jnp.float32),
        pltpu.VMEM((NW * NW + L,), jnp.int32),
        pltpu.VMEM((BKT,), jnp.int32),
        pltpu.VMEM((2 * L,), jnp.int32),
        pltpu.VMEM((2 * L,), jnp.int32),
        pltpu.VMEM((H * 2 * L,), jnp.float32),
        pltpu.VMEM((L,), jnp.int32),
        pltpu.VMEM((H * L,), jnp.float32),
        pltpu.VMEM((L, D), jnp.float32),
        pltpu.SemaphoreType.DMA,
    ],
)
def _sc_pass2(binned_hbm, cnts_hbm, dpart_hbm, xlin_hbm, a_hbm,
              acc_v, rd_v, dtmp_v, cnts_v, bkt_v, wls_v, wlb_v, wlq_v,
              pb2_v, pq2_v, rows_v, sem_r):
    w = _wid()
    lo = w * N_TILE
    zero = jnp.zeros((L,), jnp.float32)

    def zr(i, _):
        rd_v[pl.ds(i * L, L)] = zero
        return 0

    lax.fori_loop(0, (N_TILE * H) // L, zr, 0)

    def red(t, _):
        pltpu.sync_copy(
            dpart_hbm.at[pl.ds(t * (N_PAD * H) + lo * H, N_TILE * H)], dtmp_v)

        def addg(i, _):
            sl = pl.ds(i * L, L)
            rd_v[sl] = rd_v[sl] + dtmp_v[sl]
            return 0

        lax.fori_loop(0, (N_TILE * H) // L, addg, 0)
        return 0

    lax.fori_loop(0, NW, red, 0)

    def recip(i, _):
        sl = pl.ds(i * L, L)
        dn = rd_v[sl]
        rd_v[sl] = jnp.where(dn > 0, 1.0 / dn, 0.0)
        return 0

    lax.fori_loop(0, (N_TILE * H) // L, recip, 0)

    def z(i, _):
        acc_v[pl.ds(i * L, L)] = zero
        return 0

    lax.fori_loop(0, (N_TILE * H * D) // L, z, 0)
    iota = jnp.arange(L, dtype=jnp.int32)
    pltpu.sync_copy(cnts_hbm, cnts_v)

    def fire_pending():
        idx16 = wls_v[pl.ds(0, L)]
        pltpu.async_copy(xlin_hbm.at[idx16], rows_v, sem_r)
        pb2_v[pl.ds(0, L)] = wlb_v[pl.ds(0, L)]
        for h in range(H):
            pq2_v[pl.ds(h * L, L)] = wlq_v[pl.ds(h * 2 * L, L)]

    def acc_pending():
        pltpu.make_async_copy(xlin_hbm.at[pl.ds(0, L)], rows_v, sem_r).wait()
        bvec = pb2_v[pl.ds(0, L)]
        qvecs = [pq2_v[pl.ds(h * L, L)] for h in range(H)]
        for i in range(L):
            rvecs = [rows_v[i, pl.ds(cg * L, L)] for cg in range(D // L)]
            bi = bvec[i]
            for h in range(H):
                qh = qvecs[h][i]
                for cg in range(D // L):
                    plsc.addupdate(acc_v.at[pl.ds(bi + h * D + cg * L, L)],
                                   qh * rvecs[cg])

    def shift_fwd():
        for ref in (wls_v, wlb_v):
            v = ref[pl.ds(L, L)]
            ref[pl.ds(0, L)] = v
        for h in range(H):
            v = wlq_v[pl.ds(h * 2 * L + L, L)]
            wlq_v[pl.ds(h * 2 * L, L)] = v

    def bucket(wsrc, carry):
        off0, pend0 = carry
        pltpu.sync_copy(binned_hbm.at[pl.ds(wsrc * TREG + w * BKT, BKT)],
                        bkt_v)
        n = jnp.minimum(cnts_v[pl.ds(wsrc * NW + w, L)][0], CAP)

        def grp(g, c):
            off, pend = c
            mask = (g * L + iota) < n
            cnt = jnp.minimum(jnp.maximum(n - g * L, 0), L)

            @pl.when(cnt > 0)
            def _():
                s16 = bkt_v[pl.ds(g * L, L)]
                s16 = jnp.minimum(jnp.maximum(s16, 0), N_NODES - 1)
                b16 = bkt_v[pl.ds(CAP + g * L, L)]
                b16 = jnp.minimum(jnp.maximum(b16, 0),
                                  (N_TILE - 1) * (H * D))
                plsc.store_compressed(wls_v.at[pl.ds(off, L)], s16, mask=mask)
                plsc.store_compressed(wlb_v.at[pl.ds(off, L)], b16, mask=mask)
                ra = b16 >> 7
                for h in range(H):
                    p16 = plsc.bitcast(bkt_v[pl.ds((2 + h) * CAP + g * L, L)],
                                       jnp.float32)
                    r16 = plsc.load_gather(rd_v, [ra + h])
                    plsc.store_compressed(wlq_v.at[pl.ds(h * 2 * L + off, L)],
                                          p16 * r16, mask=mask)

            noff = off + cnt

            @pl.when(noff >= L)
            def _():
                @pl.when(pend == 1)
                def _():
                    acc_pending()

                fire_pending()
                shift_fwd()

            return (jnp.where(noff >= L, noff - L, noff),
                    jnp.where(noff >= L, 1, pend))

        return lax.fori_loop(0, CAP // L, grp, (off0, pend0))

    rem, pend = lax.fori_loop(0, NW, bucket, (jnp.int32(0), jnp.int32(0)))

    @pl.when(pend == 1)
    def _():
        acc_pending()

    @pl.when(rem > 0)
    def _():
        m = iota < rem
        wls_v[pl.ds(0, L)] = jnp.where(m, wls_v[pl.ds(0, L)], iota)
        wlb_v[pl.ds(0, L)] = jnp.where(m, wlb_v[pl.ds(0, L)], 0)
        for h in range(H):
            sl = pl.ds(h * 2 * L, L)
            wlq_v[sl] = jnp.where(m, wlq_v[sl], 0.0)
        fire_pending()
        acc_pending()

    pltpu.sync_copy(acc_v, a_hbm.at[pl.ds(w * N_TILE * H * D, N_TILE * H * D)])


def _prep_body(x_ref, wlin_ref, wh_ref, asrc_ref, adst_ref,
               xlin_ref, sa_ref, bmax_ref):
    i = pl.program_id(0)
    xlin = jnp.dot(x_ref[...], wlin_ref[...], preferred_element_type=jnp.float32)
    xlin_ref[...] = xlin
    cols = []
    for h in range(H):
        cols.append(jnp.dot(wh_ref[h], asrc_ref[h], preferred_element_type=jnp.float32))
    cols.append(jnp.zeros((D,), jnp.float32))
    for h in range(H):
        cols.append(jnp.dot(wh_ref[h], adst_ref[h], preferred_element_type=jnp.float32))
    cols.append(jnp.zeros((D,), jnp.float32))
    u8 = jnp.stack(cols, axis=1)
    sa = jnp.dot(xlin, u8, preferred_element_type=jnp.float32)
    sa_ref[...] = sa
    blk_max = (jnp.max(sa[:, :3]) + jnp.max(sa[:, 4:7])).reshape(1, 1)

    @pl.when(i == 0)
    def _():
        bmax_ref[...] = blk_max

    @pl.when(i > 0)
    def _():
        bmax_ref[...] = jnp.maximum(bmax_ref[...], blk_max)


def _tc_prep(x, W_lin, W_heads, att_src, att_dst):
    blk = 1000
    return pl.pallas_call(
        _prep_body,
        grid=(N_NODES // blk,),
        in_specs=[
            pl.BlockSpec((blk, D), lambda i: (i, 0)),
            pl.BlockSpec((D, D), lambda i: (0, 0)),
            pl.BlockSpec((H, D, D), lambda i: (0, 0, 0)),
            pl.BlockSpec((H, D), lambda i: (0, 0)),
            pl.BlockSpec((H, D), lambda i: (0, 0)),
        ],
        out_specs=[
            pl.BlockSpec((blk, D), lambda i: (i, 0)),
            pl.BlockSpec((blk, 8), lambda i: (i, 0)),
            pl.BlockSpec((1, 1), lambda i: (0, 0)),
        ],
        out_shape=[
            jax.ShapeDtypeStruct((N_NODES, D), jnp.float32),
            jax.ShapeDtypeStruct((N_NODES, 8), jnp.float32),
            jax.ShapeDtypeStruct((1, 1), jnp.float32),
        ],
    )(x, W_lin, W_heads, att_src, att_dst)


def _tscore_body(emb_ref, we_ref, ae_ref, t_ref, tmax_ref):
    i = pl.program_id(0)
    cols = []
    for h in range(H):
        cols.append(jnp.dot(we_ref[h], ae_ref[h], preferred_element_type=jnp.float32))
    cols.append(jnp.zeros((D,), jnp.float32))
    v4 = jnp.stack(cols, axis=1)
    t = jnp.dot(emb_ref[...], v4, preferred_element_type=jnp.float32)
    t_ref[...] = t
    blk_max = jnp.max(t[:, :3]).reshape(1, 1)

    @pl.when(i == 0)
    def _():
        tmax_ref[...] = blk_max

    @pl.when(i > 0)
    def _():
        tmax_ref[...] = jnp.maximum(tmax_ref[...], blk_max)


def _tc_tscore(emb_pad, W_edge, att_edge):
    blk = 512
    return pl.pallas_call(
        _tscore_body,
        grid=(EMB_PAD // blk,),
        in_specs=[
            pl.BlockSpec((blk, D), lambda i: (i, 0)),
            pl.BlockSpec((H, D, D), lambda i: (0, 0, 0)),
            pl.BlockSpec((H, D), lambda i: (0, 0)),
        ],
        out_specs=[
            pl.BlockSpec((blk, 4), lambda i: (i, 0)),
            pl.BlockSpec((1, 1), lambda i: (0, 0)),
        ],
        out_shape=[
            jax.ShapeDtypeStruct((EMB_PAD, 4), jnp.float32),
            jax.ShapeDtypeStruct((1, 1), jnp.float32),
        ],
    )(emb_pad, W_edge, att_edge)


def _combine_body(a_ref, wh_ref, bias_ref, out_ref):
    acc = jnp.zeros(out_ref.shape, jnp.float32)
    for h in range(H):
        acc += jnp.dot(a_ref[:, h, :], wh_ref[h], preferred_element_type=jnp.float32)
    out_ref[...] = acc * (1.0 / H) + bias_ref[...]


def _tc_combine(a_acc, W_heads, bias_mean):
    blk = 1024
    return pl.pallas_call(
        _combine_body,
        grid=(N_PAD // blk,),
        in_specs=[
            pl.BlockSpec((blk, H, D), lambda i: (i, 0, 0)),
            pl.BlockSpec((H, D, D), lambda i: (0, 0, 0)),
            pl.BlockSpec((1, D), lambda i: (0, 0)),
        ],
        out_specs=pl.BlockSpec((blk, D), lambda i: (i, 0)),
        out_shape=jax.ShapeDtypeStruct((N_PAD, D), jnp.float32),
    )(a_acc, W_heads, bias_mean)


def kernel(x, edge_index, edge_weight, W_lin, edge_emb, W_heads, att_src,
           att_dst, W_edge, att_edge, bias):
    src = edge_index[0].astype(jnp.int32)
    dst = edge_index[1].astype(jnp.int32)
    ew = edge_weight.astype(jnp.int32)

    x_lin, sa, b_sa = _tc_prep(x, W_lin, W_heads, att_src, att_dst)
    emb_pad = jnp.pad(edge_emb, ((0, EMB_PAD - EMB_ROWS), (0, 0)))
    t_tab, b_t = _tc_tscore(emb_pad, W_edge, att_edge)
    bvec = jnp.full((L,), b_sa[0, 0] + b_t[0, 0], jnp.float32)

    asd = _sc_pass1a(sa.reshape(-1), src, dst)
    p_flat, dpart = _sc_pass1b(t_tab.reshape(-1), ew, dst, asd, bvec)
    dsrec = jnp.concatenate([dst.reshape(NCH2, 1, CH2),
                             src.reshape(NCH2, 1, CH2)], axis=1).reshape(-1)
    binned, cnts = _sc_pass1c(dsrec, p_flat)
    a_flat = _sc_pass2(binned, cnts, dpart, x_lin)
    a_acc = a_flat.reshape(N_PAD, H, D)

    bias_mean = jnp.mean(bias, axis=0, keepdims=True)
    out = _tc_combine(a_acc, W_heads, bias_mean)
    return out[:N_NODES]
